# baseline (speedup 1.0000x reference)
import jax
import jax.numpy as jnp
from jax import lax
from jax.experimental import pallas as pl
from jax.experimental.pallas import tpu as pltpu
from jax.experimental.pallas import tpu_sc as plsc

_N = 100000
_E = 1600000
_G = 1024
_NPAD = 100352
_RNG = 12544

_f32 = jnp.float32
_i32 = jnp.int32


def _sds(shape, dtype=_f32):
    return jax.ShapeDtypeStruct(shape, dtype)


_MESH = plsc.VectorSubcoreMesh(core_axis_name="c", subcore_axis_name="s",
                               num_cores=2, num_subcores=16)
_SC_PARAMS = pltpu.CompilerParams(needs_layout_passes=False,
                                  use_tc_tiling_on_sc=False)


def _seg_sum_pass(src_hbm, dst_hbm, tab_hbm, out_hbm, lo, s,
                  sbuf, dbuf, gidx, sidx, g32, s32, rows, rows32,
                  zbuf, wbuf, accum, sem, code_v=None):
    def _zero(k, _):
        idx = s + k * 16

        @pl.when(idx < 99)
        def _():
            pltpu.sync_copy(zbuf, accum.at[pl.ds(idx * 128, 128)])
        return 0
    lax.fori_loop(0, 7, _zero, 0)
    plsc.subcore_barrier()

    base = s * 100000

    def _edges16(sl, gdst, sdst, i):
        s16 = sbuf[sl]
        d16 = dbuf[sl]
        if code_v is not None:
            g16 = plsc.load_gather(code_v, [s16])
        else:
            g16 = s16
        inr = (d16 >= lo) & (d16 < lo + _RNG)
        idx16 = jnp.where(inr, d16 - lo, _RNG)
        gdst[pl.ds(i * 16, 16)] = g16
        sdst[pl.ds(i * 16, 16)] = idx16

    def _sub(k, _):
        for i in range(8):
            _edges16(pl.ds(k * 128 + i * 16, 16), gidx, sidx, i)
        pltpu.async_copy(tab_hbm.at[gidx], rows, sem).wait()
        pltpu.sync_copy(rows, accum.at[sidx], add=True)
        return 0

    def _chunk(j, _):
        off = base + j * 1024
        pltpu.sync_copy(src_hbm.at[pl.ds(off, 1024)], sbuf)
        pltpu.sync_copy(dst_hbm.at[pl.ds(off, 1024)], dbuf)
        lax.fori_loop(0, 8, _sub, 0)
        return 0
    lax.fori_loop(0, 97, _chunk, 0)

    toff = base + 99328
    pltpu.sync_copy(src_hbm.at[pl.ds(toff, 672)], sbuf.at[pl.ds(0, 672)])
    pltpu.sync_copy(dst_hbm.at[pl.ds(toff, 672)], dbuf.at[pl.ds(0, 672)])
    lax.fori_loop(0, 5, _sub, 0)
    for i in range(2):
        _edges16(pl.ds(640 + i * 16, 16), g32, s32, i)
    pltpu.async_copy(tab_hbm.at[g32], rows32, sem).wait()
    pltpu.sync_copy(rows32, accum.at[s32], add=True)
    plsc.subcore_barrier()

    def _wb(k, _):
        idx = s + k * 16

        @pl.when(idx < 98)
        def _():
            pltpu.sync_copy(accum.at[pl.ds(idx * 128, 128)], wbuf)
            pltpu.sync_copy(wbuf, out_hbm.at[pl.ds(lo + idx * 128, 128)])
        return 0
    lax.fori_loop(0, 7, _wb, 0)
    plsc.subcore_barrier()


def _sc_layer1(src_hbm, dst_hbm, code_hbm, t0a_hbm, t0b_hbm, tR_hbm, z_hbm,
               a1a_hbm, a1b_hbm, root_hbm,
               code_v, sbuf, dbuf, gidx, sidx, g32, s32, rows, rows32,
               zbuf, wbuf, crows, cidx, accum, sem):
    c = lax.axis_index("c")
    s = lax.axis_index("s")

    pltpu.sync_copy(code_hbm, code_v)
    pltpu.sync_copy(z_hbm, zbuf)

    for tab, out in ((t0a_hbm, a1a_hbm), (t0b_hbm, a1b_hbm)):
        for p in range(4):
            lo = (c * 4 + p) * _RNG
            _seg_sum_pass(src_hbm, dst_hbm, tab, out, lo, s,
                          sbuf, dbuf, gidx, sidx, g32, s32, rows, rows32,
                          zbuf, wbuf, accum, sem, code_v=code_v)

    w = c * 16 + s

    def _root(k, _):
        idx = w + k * 32
        off = idx * 128
        pltpu.sync_copy(code_hbm.at[pl.ds(off, 128)], cidx)
        pltpu.async_copy(tR_hbm.at[cidx], crows, sem).wait()
        pltpu.sync_copy(crows, root_hbm.at[pl.ds(off, 128)])
        return 0
    lax.fori_loop(0, 24, _root, 0)

    @pl.when(w < 16)
    def _():
        off = (768 + w) * 128
        pltpu.sync_copy(code_hbm.at[pl.ds(off, 128)], cidx)
        pltpu.async_copy(tR_hbm.at[cidx], crows, sem).wait()
        pltpu.sync_copy(crows, root_hbm.at[pl.ds(off, 128)])


def _sc_layer2(src_hbm, dst_hbm, h0_hbm, h1_hbm, h2_hbm, h3_hbm, z_hbm,
               o0_hbm, o1_hbm, o2_hbm, o3_hbm,
               sbuf, dbuf, gidx, sidx, g32, s32, rows, rows32,
               zbuf, wbuf, accum, sem):
    c = lax.axis_index("c")
    s = lax.axis_index("s")
    pltpu.sync_copy(z_hbm, zbuf)

    for tab, out in ((h0_hbm, o0_hbm), (h1_hbm, o1_hbm),
                     (h2_hbm, o2_hbm), (h3_hbm, o3_hbm)):
        for p in range(4):
            lo = (c * 4 + p) * _RNG
            _seg_sum_pass(src_hbm, dst_hbm, tab, out, lo, s,
                          sbuf, dbuf, gidx, sidx, g32, s32, rows, rows32,
                          zbuf, wbuf, accum, sem)


def _tc_pool(b_ref, h_ref, o_ref):
    i = pl.program_id(0)
    b_blk = b_ref[0, 0, :]
    giota = lax.broadcasted_iota(_i32, (_G, 2048), 0)
    oh = (b_blk[None, :] == giota).astype(_f32)
    part = jnp.dot(oh, h_ref[...], preferred_element_type=_f32)

    @pl.when(i == 0)
    def _():
        o_ref[...] = jnp.zeros_like(o_ref)
    o_ref[...] += part


def _tc_tables(cat_ref, wl_ref, bl_ref, wr_ref, t0_ref, tR_ref):
    t0 = jax.nn.relu(jnp.dot(cat_ref[...], wl_ref[...],
                             preferred_element_type=_f32) + bl_ref[...])
    t0_ref[...] = t0
    tR_ref[...] = jnp.dot(t0, wr_ref[...], preferred_element_type=_f32)


def _tc_h1(aa_ref, ab_ref, r_ref, w_ref, b_ref, o0_ref, o1_ref, o2_ref,
           o3_ref):
    h = jax.nn.relu(
        jnp.dot(aa_ref[...], w_ref[:16, :], preferred_element_type=_f32)
        + jnp.dot(ab_ref[...], w_ref[16:, :], preferred_element_type=_f32)
        + r_ref[...] + b_ref[...])
    o0_ref[...] = h[:, :16]
    o1_ref[...] = h[:, 16:32]
    o2_ref[...] = h[:, 32:48]
    o3_ref[...] = h[:, 48:]


def _tc_h2(a0_ref, a1_ref, a2_ref, a3_ref, h0_ref, h1_ref, h2_ref, h3_ref,
           wr_ref, wo_ref, b_ref, o_ref):
    acc = b_ref[...]
    for q, (a, h) in enumerate(((a0_ref, h0_ref), (a1_ref, h1_ref),
                                (a2_ref, h2_ref), (a3_ref, h3_ref))):
        sl = slice(q * 16, q * 16 + 16)
        acc = acc + jnp.dot(a[...], wr_ref[sl, :],
                            preferred_element_type=_f32)
        acc = acc + jnp.dot(h[...], wo_ref[sl, :],
                            preferred_element_type=_f32)
    v = jax.nn.relu(acc)
    lane = lax.broadcasted_iota(_i32, (v.shape[0], 16), 1)
    aug = jnp.where(lane == 0, 1.0, 0.0).astype(_f32)
    o_ref[...] = jnp.concatenate([v, aug], axis=1)


def _tc_cls(p_ref, w_ref, b_ref, o_ref):
    ptot = p_ref[...]
    sums = ptot[:, :64]
    cnt = jnp.maximum(ptot[:, 64:65], 1.0)
    pooled = sums / cnt
    o_ref[...] = jnp.dot(pooled, w_ref[...],
                         preferred_element_type=_f32) + b_ref[...]


def kernel(x, edge_index, batch, shape_emb, color_emb, W_lin, b_lin,
           W_rel1, b_rel1, W_root1, W_rel2, b_rel2, W_root2, W_cls, b_cls):
    code = jnp.pad((x[:, 0] * 16 + x[:, 1]).astype(_i32),
                   (0, _NPAD - _N))
    src = edge_index[0].astype(_i32)
    dst = edge_index[1].astype(_i32)
    cat = jnp.concatenate(
        [jnp.repeat(shape_emb, 16, axis=0), jnp.tile(color_emb, (16, 1))],
        axis=1)

    table0, tableR = pl.pallas_call(
        _tc_tables,
        out_shape=(_sds((256, 32)), _sds((256, 64))),
    )(cat, W_lin, b_lin[None, :], W_root1)
    t0a = table0[:, :16]
    t0b = table0[:, 16:]

    z16 = jnp.zeros((128, 16), _f32)

    layer1 = pl.kernel(
        _sc_layer1,
        out_type=(_sds((_NPAD, 16)), _sds((_NPAD, 16)), _sds((_NPAD, 64))),
        mesh=_MESH,
        scratch_types=[
            pltpu.VMEM((_NPAD,), _i32),
            pltpu.VMEM((1024,), _i32),
            pltpu.VMEM((1024,), _i32),
            pltpu.VMEM((128,), _i32),
            pltpu.VMEM((128,), _i32),
            pltpu.VMEM((32,), _i32),
            pltpu.VMEM((32,), _i32),
            pltpu.VMEM((128, 16), _f32),
            pltpu.VMEM((32, 16), _f32),
            pltpu.VMEM((128, 16), _f32),
            pltpu.VMEM((128, 16), _f32),
            pltpu.VMEM((128, 64), _f32),
            pltpu.VMEM((128,), _i32),
            pltpu.VMEM_SHARED((12672, 16), _f32),
            pltpu.SemaphoreType.DMA,
        ],
        compiler_params=_SC_PARAMS)
    a1a, a1b, root1n = layer1(src, dst, code, t0a, t0b, tableR, z16)

    h1q = pl.pallas_call(
        _tc_h1,
        grid=(49,),
        in_specs=[
            pl.BlockSpec((2048, 16), lambda i: (i, 0)),
            pl.BlockSpec((2048, 16), lambda i: (i, 0)),
            pl.BlockSpec((2048, 64), lambda i: (i, 0)),
            pl.BlockSpec((32, 64), lambda i: (0, 0)),
            pl.BlockSpec((1, 64), lambda i: (0, 0)),
        ],
        out_specs=tuple(pl.BlockSpec((2048, 16), lambda i: (i, 0))
                        for _ in range(4)),
        out_shape=tuple(_sds((_NPAD, 16)) for _ in range(4)),
    )(a1a, a1b, root1n, W_rel1, b_rel1[None, :])

    layer2 = pl.kernel(
        _sc_layer2,
        out_type=tuple(_sds((_NPAD, 16)) for _ in range(4)),
        mesh=_MESH,
        scratch_types=[
            pltpu.VMEM((1024,), _i32),
            pltpu.VMEM((1024,), _i32),
            pltpu.VMEM((128,), _i32),
            pltpu.VMEM((128,), _i32),
            pltpu.VMEM((32,), _i32),
            pltpu.VMEM((32,), _i32),
            pltpu.VMEM((128, 16), _f32),
            pltpu.VMEM((32, 16), _f32),
            pltpu.VMEM((128, 16), _f32),
            pltpu.VMEM((128, 16), _f32),
            pltpu.VMEM_SHARED((12672, 16), _f32),
            pltpu.SemaphoreType.DMA,
        ],
        compiler_params=_SC_PARAMS)
    agg2q = layer2(src, dst, *h1q, z16)

    h2aug = pl.pallas_call(
        _tc_h2,
        grid=(49,),
        in_specs=(
            [pl.BlockSpec((2048, 16), lambda i: (i, 0)) for _ in range(8)]
            + [pl.BlockSpec((64, 64), lambda i: (0, 0)),
               pl.BlockSpec((64, 64), lambda i: (0, 0)),
               pl.BlockSpec((1, 64), lambda i: (0, 0))]),
        out_specs=pl.BlockSpec((2048, 80), lambda i: (i, 0)),
        out_shape=_sds((_NPAD, 80)),
    )(*agg2q, *h1q, W_rel2, W_root2, b_rel2[None, :])

    batch3 = jnp.pad(batch.astype(_i32), (0, _NPAD - _N),
                     constant_values=_G).reshape(49, 1, 2048)
    pooled = pl.pallas_call(
        _tc_pool,
        grid=(49,),
        in_specs=[
            pl.BlockSpec((1, 1, 2048), lambda i: (i, 0, 0)),
            pl.BlockSpec((2048, 80), lambda i: (i, 0)),
        ],
        out_specs=pl.BlockSpec((_G, 80), lambda i: (0, 0)),
        out_shape=_sds((_G, 80)),
    )(batch3, h2aug)

    Wp = jnp.pad(W_cls, ((0, 0), (0, 128 - W_cls.shape[1])))
    bp = jnp.pad(b_cls, (0, 128 - b_cls.shape[0]))[None, :]
    out = pl.pallas_call(
        _tc_cls,
        out_shape=_sds((_G, 128)),
    )(pooled, Wp, bp)
    return out[:, :W_cls.shape[1]]
